# baseline (speedup 1.0000x reference)
import functools

import jax
import jax.numpy as jnp
import numpy as np
from jax import lax
from jax.experimental import pallas as pl
from jax.experimental.pallas import tpu as pltpu
from jax.experimental.pallas import tpu_sc as plsc

B, F, D, V = 16384, 26, 16, 1000000
FD = F * D
BF = B * F

NC, NS, L = 2, 16, 16
NW = NC * NS
PW = BF // NW

_TCH = 62504


@functools.cache
def _make_sc_gather():
    mesh = plsc.VectorSubcoreMesh(
        core_axis_name="c", subcore_axis_name="s", num_cores=NC, num_subcores=NS
    )
    return pl.kernel(
        _sc_gather_body,
        out_type=jax.ShapeDtypeStruct((BF,), jnp.float32),
        mesh=mesh,
        scratch_types=[
            pltpu.VMEM((PW,), jnp.int32),
            pltpu.VMEM((PW,), jnp.float32),
            pltpu.VMEM_SHARED((V,), jnp.float32),
            pltpu.VMEM((8192,), jnp.float32),
            pltpu.SemaphoreType.DMA,
        ],
    )


def _sc_gather_body(table_hbm, ids_hbm, out_hbm, idx_v, g_v, table_sh,
                    bounce_v, sem):
    i32 = jnp.int32
    sid = lax.axis_index("s")
    wid = sid * i32(NC) + lax.axis_index("c")
    base = wid * i32(PW)

    idx_desc = pltpu.async_copy(ids_hbm.at[pl.ds(base, PW)], idx_v, sem)

    toff = jnp.minimum(sid * i32(_TCH), i32(V - _TCH))
    sub_off = 0
    for sz in [8192] * 7 + [_TCH - 7 * 8192]:
        src = table_hbm.at[pl.ds(toff + i32(sub_off), sz)]
        pltpu.sync_copy(src, bounce_v.at[pl.ds(i32(0), sz)])
        pltpu.sync_copy(bounce_v.at[pl.ds(i32(0), sz)],
                        table_sh.at[pl.ds(toff + i32(sub_off), sz)])
        sub_off += sz
    plsc.subcore_barrier()

    idx_desc.wait()
    pltpu.async_copy(table_sh.at[idx_v], g_v, sem).wait()
    pltpu.sync_copy(g_v, out_hbm.at[pl.ds(base, PW)])


_BB = 2048


def _tc_body(x_ref, m_ref, g_ref, v_ref, b_ref, o_ref):
    x = x_ref[...]
    m = m_ref[...]
    s = jnp.dot(x, m, preferred_element_type=jnp.float32)
    t1 = jnp.sum(s * s, axis=1)
    t2 = jnp.sum(x * x, axis=1)
    t3 = jnp.sum(g_ref[...] * v_ref[...], axis=1)
    o_ref[...] = (t1 - t2) * (1.0 / (2.0 * D)) + t3 + b_ref[0]


_tc_call = pl.pallas_call(
    _tc_body,
    out_shape=jax.ShapeDtypeStruct((B,), jnp.float32),
    grid=(B // _BB,),
    in_specs=[
        pl.BlockSpec((_BB, FD), lambda i: (i, jnp.int32(0))),
        pl.BlockSpec((FD, D), lambda i: (jnp.int32(0), jnp.int32(0))),
        pl.BlockSpec((_BB, F), lambda i: (i, jnp.int32(0))),
        pl.BlockSpec((_BB, F), lambda i: (i, jnp.int32(0))),
        pl.BlockSpec((1,), lambda i: (jnp.int32(0),), memory_space=pltpu.SMEM),
    ],
    out_specs=pl.BlockSpec((_BB,), lambda i: (i,)),
)

_M_np = np.zeros((FD, D), dtype=np.float32)
_M_np[np.arange(FD), np.arange(FD) % D] = 1.0


@jax.jit
def kernel(input_embeddings, feature_ids, feature_vals, feature_biases, bias):
    ids = feature_ids.reshape(BF).astype(jnp.int32)
    table = feature_biases.reshape(V)
    g = _make_sc_gather()(table, ids).reshape(B, F)

    x = input_embeddings.reshape(B, FD)
    m = jnp.asarray(_M_np)
    return _tc_call(x, m, g, feature_vals, bias.astype(jnp.float32))
